# baseline (speedup 1.0000x reference)
import functools

import jax
import jax.numpy as jnp
from jax import lax
from jax.experimental import pallas as pl
from jax.experimental.pallas import tpu as pltpu
from jax.experimental.pallas import tpu_sc as plsc

N_TYPES = 8
D = 128
N = 100000
BLK = 10000
N_BLOCKS = N // BLK

NW = 32
NP = 102400
CH = NP // NW
NVREG = CH // 16
NGRP = CH // 128
G_FULL = N // 128
REM = N - G_FULL * 128
REM_OFF = G_FULL * 128
SENTINEL = 127

_MESH = dict(core_axis_name="c", subcore_axis_name="s")


def _wid():
    return lax.axis_index("s") * 2 + lax.axis_index("c")


def _hist_body(ids_hbm, counts_hbm, idsv, cntv):
    w = _wid()
    pltpu.sync_copy(ids_hbm.at[pl.ds(w * CH, CH)], idsv)
    lane = lax.iota(jnp.int32, 16)
    zeros = jnp.zeros((16,), jnp.int32)

    one = jnp.ones((16,), jnp.int32)

    def body(i, cs):
        vec = idsv[pl.ds(i * 16, 16)]
        return tuple(
            cs[t] + jnp.where(vec == t, one, zeros) for t in range(N_TYPES)
        )

    cs = lax.fori_loop(0, NVREG, body, (zeros,) * N_TYPES)
    cv = zeros
    for t in range(N_TYPES):
        tot = jnp.int32(0)
        for l in range(16):
            tot = tot + cs[t][l]
        cv = jnp.where(lane == t, tot, cv)
    cntv[...] = cv
    pltpu.sync_copy(cntv, counts_hbm.at[w])


def _hist(ids_pad):
    return pl.kernel(
        _hist_body,
        out_type=jax.ShapeDtypeStruct((NW, 16), jnp.int32),
        mesh=plsc.VectorSubcoreMesh(**_MESH),
        name="sc_hist",
        scratch_types=[
            pltpu.VMEM((CH,), jnp.int32),
            pltpu.VMEM((16,), jnp.int32),
        ],
    )(ids_pad)


def _dispatch_body(ids_hbm, counts_hbm, inv_hbm, bases_hbm,
                   idsv, cntall, destv, basev):
    w = _wid()
    pltpu.sync_copy(ids_hbm.at[pl.ds(w * CH, CH)], idsv)
    pltpu.sync_copy(counts_hbm, cntall)
    lane = lax.iota(jnp.int32, 16)
    zeros = jnp.zeros((16,), jnp.int32)

    rows = [cntall[wp] for wp in range(NW)]
    base = jnp.int32(0)
    bv = zeros
    offs0 = []
    for t in range(N_TYPES):
        myprefix = jnp.int32(0)
        total = jnp.int32(0)
        for wp in range(NW):
            c = rows[wp][t]
            total = total + c
            myprefix = myprefix + jnp.where(jnp.int32(wp) < w, c, 0)
        bv = jnp.where(lane == t, base, bv)
        offs0.append(base + myprefix)
        base = base + total
    offs0.append(jnp.int32(N))
    bv = jnp.where(lane >= N_TYPES, jnp.int32(N), bv)
    basev[...] = bv

    @pl.when(w == 0)
    def _():
        pltpu.sync_copy(basev, bases_hbm)

    one = jnp.ones((16,), jnp.int32)
    lane_gt = [lane > k for k in range(16)]

    def body(i, offs):
        vec = idsv[pl.ds(i * 16, 16)]
        tidv = jnp.minimum(vec, N_TYPES)
        tids = [tidv[k] for k in range(16)]
        r = zeros
        for k in range(16):
            r = r + jnp.where(lane_gt[k] & (tidv == tids[k]), one, zeros)
        offv = zeros
        for t in range(N_TYPES + 1):
            offv = offv + jnp.where(tidv == t, offs[t], 0)
        destv[pl.ds(i * 16, 16)] = offv + r
        new_offs = []
        for t in range(N_TYPES + 1):
            c = jnp.int32(0)
            for k in range(16):
                c = c + jnp.where(tids[k] == t, 1, 0)
            new_offs.append(offs[t] + c)
        return tuple(new_offs)

    lax.fori_loop(0, NVREG, body, tuple(offs0))
    pltpu.sync_copy(destv, inv_hbm.at[pl.ds(w * CH, CH)])


def _dispatch(ids_pad, counts):
    return pl.kernel(
        _dispatch_body,
        out_type=[
            jax.ShapeDtypeStruct((NP,), jnp.int32),
            jax.ShapeDtypeStruct((16,), jnp.int32),
        ],
        mesh=plsc.VectorSubcoreMesh(**_MESH),
        name="sc_dispatch",
        scratch_types=[
            pltpu.VMEM((CH,), jnp.int32),
            pltpu.VMEM((NW, 16), jnp.int32),
            pltpu.VMEM((CH,), jnp.int32),
            pltpu.VMEM((16,), jnp.int32),
        ],
    )(ids_pad, counts)


def _gather_body(inv_hbm, h_hbm, m_hbm, hs_hbm, ms_hbm,
                 idxb, hbuf, mbuf, idxrem, hrem, mrem, isem, lsem, ssem):
    w = _wid()
    gbase = w * NGRP

    def valid(s):
        return (s >= 0) & (s < NGRP) & (gbase + s < G_FULL)

    def body(slot, _):
        s1 = slot - 1
        s3 = slot - 3

        @pl.when(valid(s3))
        def _():
            pltpu.make_async_copy(
                hbuf.at[s3 % 3], hs_hbm.at[idxb.at[s3 % 3]], ssem).wait()
            pltpu.make_async_copy(
                mbuf.at[s3 % 3], ms_hbm.at[idxb.at[s3 % 3]], ssem).wait()

        @pl.when(valid(slot))
        def _():
            off = (gbase + slot) * 128
            pltpu.async_copy(
                inv_hbm.at[pl.ds(off, 128)], idxb.at[slot % 3], isem)
            pltpu.async_copy(
                h_hbm.at[pl.ds(off, 128)], hbuf.at[slot % 3], lsem)
            pltpu.async_copy(
                m_hbm.at[pl.ds(off, 128)], mbuf.at[slot % 3], lsem)

        @pl.when(valid(s1))
        def _():
            off = (gbase + s1) * 128
            pltpu.make_async_copy(
                inv_hbm.at[pl.ds(off, 128)], idxb.at[s1 % 3], isem).wait()
            pltpu.make_async_copy(
                h_hbm.at[pl.ds(off, 128)], hbuf.at[s1 % 3], lsem).wait()
            pltpu.make_async_copy(
                m_hbm.at[pl.ds(off, 128)], mbuf.at[s1 % 3], lsem).wait()
            pltpu.async_copy(hbuf.at[s1 % 3], hs_hbm.at[idxb.at[s1 % 3]], ssem)
            pltpu.async_copy(mbuf.at[s1 % 3], ms_hbm.at[idxb.at[s1 % 3]], ssem)

        return 0

    lax.fori_loop(0, NGRP + 4, body, 0)

    @pl.when(w == 0)
    def _():
        pltpu.sync_copy(inv_hbm.at[pl.ds(REM_OFF, REM)], idxrem)
        pltpu.sync_copy(h_hbm.at[pl.ds(REM_OFF, REM)], hrem)
        pltpu.sync_copy(m_hbm.at[pl.ds(REM_OFF, REM)], mrem)
        pltpu.async_copy(hrem, hs_hbm.at[idxrem], ssem).wait()
        pltpu.async_copy(mrem, ms_hbm.at[idxrem], ssem).wait()


def _gather(inv, h, message):
    return pl.kernel(
        _gather_body,
        out_type=[
            jax.ShapeDtypeStruct((N, D), jnp.float32),
            jax.ShapeDtypeStruct((N, D), jnp.float32),
        ],
        mesh=plsc.VectorSubcoreMesh(**_MESH),
        name="sc_gather",
        scratch_types=[
            pltpu.VMEM((3, 128), jnp.int32),
            pltpu.VMEM((3, 128, D), jnp.float32),
            pltpu.VMEM((3, 128, D), jnp.float32),
            pltpu.VMEM((REM,), jnp.int32),
            pltpu.VMEM((REM, D), jnp.float32),
            pltpu.VMEM((REM, D), jnp.float32),
            pltpu.SemaphoreType.DMA,
            pltpu.SemaphoreType.DMA,
            pltpu.SemaphoreType.DMA,
        ],
    )(inv, h, message)


def _scatter_body(inv_hbm, outs_hbm, dh_hbm, idxb, rbuf, idxrem, rrem,
                  isem, gsem, wsem):
    w = _wid()
    gbase = w * NGRP

    def valid(s):
        return (s >= 0) & (s < NGRP) & (gbase + s < G_FULL)

    def body(slot, _):
        s1 = slot - 1
        s2 = slot - 2
        s4 = slot - 4

        @pl.when(valid(s4))
        def _():
            off = (gbase + s4) * 128
            pltpu.make_async_copy(
                rbuf.at[s4 % 3], dh_hbm.at[pl.ds(off, 128)], wsem).wait()

        @pl.when(valid(slot))
        def _():
            off = (gbase + slot) * 128
            pltpu.async_copy(
                inv_hbm.at[pl.ds(off, 128)], idxb.at[slot % 3], isem)

        @pl.when(valid(s1))
        def _():
            off = (gbase + s1) * 128
            pltpu.make_async_copy(
                inv_hbm.at[pl.ds(off, 128)], idxb.at[s1 % 3], isem).wait()
            pltpu.async_copy(
                outs_hbm.at[idxb.at[s1 % 3]], rbuf.at[s1 % 3], gsem)

        @pl.when(valid(s2))
        def _():
            off = (gbase + s2) * 128
            pltpu.make_async_copy(
                outs_hbm.at[idxb.at[s2 % 3]], rbuf.at[s2 % 3], gsem).wait()
            pltpu.async_copy(
                rbuf.at[s2 % 3], dh_hbm.at[pl.ds(off, 128)], wsem)

        return 0

    lax.fori_loop(0, NGRP + 5, body, 0)

    @pl.when(w == 0)
    def _():
        pltpu.sync_copy(inv_hbm.at[pl.ds(REM_OFF, REM)], idxrem)
        pltpu.async_copy(outs_hbm.at[idxrem], rrem, gsem).wait()
        pltpu.sync_copy(rrem, dh_hbm.at[pl.ds(REM_OFF, REM)])


def _scatter(inv, outs):
    return pl.kernel(
        _scatter_body,
        out_type=jax.ShapeDtypeStruct((N, D), jnp.float32),
        mesh=plsc.VectorSubcoreMesh(**_MESH),
        name="sc_scatter",
        scratch_types=[
            pltpu.VMEM((3, 128), jnp.int32),
            pltpu.VMEM((3, 128, D), jnp.float32),
            pltpu.VMEM((REM,), jnp.int32),
            pltpu.VMEM((REM, D), jnp.float32),
            pltpu.SemaphoreType.DMA,
            pltpu.SemaphoreType.DMA,
            pltpu.SemaphoreType.DMA,
        ],
    )(inv, outs)


def _sn(W):
    R = W.shape[0]
    u = jnp.full((R, 1), 1.0 / jnp.sqrt(jnp.float32(R)), dtype=jnp.float32)
    v = None
    for _ in range(7):
        v = jnp.sum(W * u, axis=0, keepdims=True)
        v = v / (jnp.sqrt(jnp.sum(v * v)) + 1e-12)
        u = jnp.sum(W * v, axis=1, keepdims=True)
        u = u / (jnp.sqrt(jnp.sum(u * u)) + 1e-12)
    Wv = jnp.sum(W * v, axis=1, keepdims=True)
    sigma = jnp.sum(u * Wv)
    return W / sigma


def _norm_body(w1_ref, w2_ref, w1o_ref, w2o_ref):
    for i in range(N_TYPES):
        w1o_ref[i] = _sn(w1_ref[i])
        w2o_ref[i] = _sn(w2_ref[i])


def _normalize(W1, W2, interpret=False):
    return pl.pallas_call(
        _norm_body,
        grid=(1,),
        in_specs=[
            pl.BlockSpec((N_TYPES, 2 * D, D), lambda i: (0, 0, 0)),
            pl.BlockSpec((N_TYPES, D, D), lambda i: (0, 0, 0)),
        ],
        out_specs=[
            pl.BlockSpec((N_TYPES, 2 * D, D), lambda i: (0, 0, 0)),
            pl.BlockSpec((N_TYPES, D, D), lambda i: (0, 0, 0)),
        ],
        out_shape=[
            jax.ShapeDtypeStruct((N_TYPES, 2 * D, D), jnp.float32),
            jax.ShapeDtypeStruct((N_TYPES, D, D), jnp.float32),
        ],
        interpret=interpret,
    )(W1, W2)


def _seg_body(bases_ref, hs_ref, ms_ref, w1_ref, b1_ref, w2_ref, b2_ref,
              out_ref):
    b = pl.program_id(0)
    start = b * BLK
    hx = hs_ref[...]
    mx = ms_ref[...]
    ridx = lax.broadcasted_iota(jnp.int32, (BLK, 1), 0) + start
    out_ref[...] = jnp.zeros((BLK, D), jnp.float32)
    for t in range(N_TYPES):
        lo = bases_ref[t]
        hi = bases_ref[t + 1]

        @pl.when((lo < start + BLK) & (hi > start))
        def _():
            y = (
                jnp.dot(hx, w1_ref[t, :D, :],
                        preferred_element_type=jnp.float32)
                + jnp.dot(mx, w1_ref[t, D:, :],
                          preferred_element_type=jnp.float32)
                + b1_ref[t]
            )
            y = y * jax.nn.sigmoid(y)
            z = jnp.dot(y, w2_ref[t], preferred_element_type=jnp.float32)
            z = z + b2_ref[t]
            mask = ((ridx >= lo) & (ridx < hi)).astype(jnp.float32)
            out_ref[...] += z * mask


def _seg_mlp(bases, hs, ms, W1n, b1, W2n, b2):
    grid_spec = pltpu.PrefetchScalarGridSpec(
        num_scalar_prefetch=1,
        grid=(N_BLOCKS,),
        in_specs=[
            pl.BlockSpec((BLK, D), lambda i, s: (i, 0)),
            pl.BlockSpec((BLK, D), lambda i, s: (i, 0)),
            pl.BlockSpec((N_TYPES, 2 * D, D), lambda i, s: (0, 0, 0)),
            pl.BlockSpec((N_TYPES, D), lambda i, s: (0, 0)),
            pl.BlockSpec((N_TYPES, D, D), lambda i, s: (0, 0, 0)),
            pl.BlockSpec((N_TYPES, D), lambda i, s: (0, 0)),
        ],
        out_specs=pl.BlockSpec((BLK, D), lambda i, s: (i, 0)),
    )
    return pl.pallas_call(
        _seg_body,
        grid_spec=grid_spec,
        out_shape=jax.ShapeDtypeStruct((N, D), jnp.float32),
        compiler_params=pltpu.CompilerParams(
            dimension_semantics=("arbitrary",),
        ),
    )(bases, hs, ms, W1n, b1, W2n, b2)


def kernel(h, message, node_type_ids, W1, b1, W2, b2):
    ids = node_type_ids.astype(jnp.int32)
    ids_pad = jnp.concatenate(
        [ids, jnp.full((NP - N,), SENTINEL, jnp.int32)])
    counts = _hist(ids_pad)
    inv, bases = _dispatch(ids_pad, counts)
    W1n, W2n = _normalize(W1, W2)
    hs, ms = _gather(inv, h, message)
    outs = _seg_mlp(bases, hs, ms, W1n, b1, W2n, b2)
    dh = _scatter(inv, outs)
    return dh


# speedup vs baseline: 1.0386x; 1.0386x over previous
import functools

import jax
import jax.numpy as jnp
from jax import lax
from jax.experimental import pallas as pl
from jax.experimental.pallas import tpu as pltpu
from jax.experimental.pallas import tpu_sc as plsc

N_TYPES = 8
D = 128
N = 100000
BLK = 5000
N_BLOCKS = N // BLK

NW = 32
NP = 102400
CH = NP // NW
NVREG = CH // 16
NGRP = CH // 128
G_FULL = N // 128
REM = N - G_FULL * 128
REM_OFF = G_FULL * 128
SENTINEL = 127

_MESH = dict(core_axis_name="c", subcore_axis_name="s")


def _wid():
    return lax.axis_index("s") * 2 + lax.axis_index("c")


def _hist_body(ids_hbm, counts_hbm, idsv, cntv):
    w = _wid()
    pltpu.sync_copy(ids_hbm.at[pl.ds(w * CH, CH)], idsv)
    lane = lax.iota(jnp.int32, 16)
    zeros = jnp.zeros((16,), jnp.int32)

    one = jnp.ones((16,), jnp.int32)

    def body(i, cs):
        vec = idsv[pl.ds(i * 16, 16)]
        return tuple(
            cs[t] + jnp.where(vec == t, one, zeros) for t in range(N_TYPES)
        )

    cs = lax.fori_loop(0, NVREG, body, (zeros,) * N_TYPES)
    cv = zeros
    for t in range(N_TYPES):
        tot = jnp.int32(0)
        for l in range(16):
            tot = tot + cs[t][l]
        cv = jnp.where(lane == t, tot, cv)
    cntv[...] = cv
    pltpu.sync_copy(cntv, counts_hbm.at[w])


def _hist(ids_pad):
    return pl.kernel(
        _hist_body,
        out_type=jax.ShapeDtypeStruct((NW, 16), jnp.int32),
        mesh=plsc.VectorSubcoreMesh(**_MESH),
        name="sc_hist",
        scratch_types=[
            pltpu.VMEM((CH,), jnp.int32),
            pltpu.VMEM((16,), jnp.int32),
        ],
    )(ids_pad)


def _dispatch_body(ids_hbm, counts_hbm, inv_hbm, bases_hbm,
                   idsv, cntall, destv, basev):
    w = _wid()
    pltpu.sync_copy(ids_hbm.at[pl.ds(w * CH, CH)], idsv)
    pltpu.sync_copy(counts_hbm, cntall)
    lane = lax.iota(jnp.int32, 16)
    zeros = jnp.zeros((16,), jnp.int32)

    rows = [cntall[wp] for wp in range(NW)]
    base = jnp.int32(0)
    bv = zeros
    offs0 = []
    for t in range(N_TYPES):
        myprefix = jnp.int32(0)
        total = jnp.int32(0)
        for wp in range(NW):
            c = rows[wp][t]
            total = total + c
            myprefix = myprefix + jnp.where(jnp.int32(wp) < w, c, 0)
        bv = jnp.where(lane == t, base, bv)
        offs0.append(base + myprefix)
        base = base + total
    offs0.append(jnp.int32(N))
    bv = jnp.where(lane >= N_TYPES, jnp.int32(N), bv)
    basev[...] = bv

    @pl.when(w == 0)
    def _():
        pltpu.sync_copy(basev, bases_hbm)

    one = jnp.ones((16,), jnp.int32)
    lane_gt = [lane > k for k in range(16)]

    def body(i, offs):
        vec = idsv[pl.ds(i * 16, 16)]
        tidv = jnp.minimum(vec, N_TYPES)
        tids = [tidv[k] for k in range(16)]
        r = zeros
        for k in range(16):
            r = r + jnp.where(lane_gt[k] & (tidv == tids[k]), one, zeros)
        offv = zeros
        for t in range(N_TYPES + 1):
            offv = offv + jnp.where(tidv == t, offs[t], 0)
        destv[pl.ds(i * 16, 16)] = offv + r
        new_offs = []
        for t in range(N_TYPES + 1):
            c = jnp.int32(0)
            for k in range(16):
                c = c + jnp.where(tids[k] == t, 1, 0)
            new_offs.append(offs[t] + c)
        return tuple(new_offs)

    lax.fori_loop(0, NVREG, body, tuple(offs0))
    pltpu.sync_copy(destv, inv_hbm.at[pl.ds(w * CH, CH)])


def _dispatch(ids_pad, counts):
    return pl.kernel(
        _dispatch_body,
        out_type=[
            jax.ShapeDtypeStruct((NP,), jnp.int32),
            jax.ShapeDtypeStruct((16,), jnp.int32),
        ],
        mesh=plsc.VectorSubcoreMesh(**_MESH),
        name="sc_dispatch",
        scratch_types=[
            pltpu.VMEM((CH,), jnp.int32),
            pltpu.VMEM((NW, 16), jnp.int32),
            pltpu.VMEM((CH,), jnp.int32),
            pltpu.VMEM((16,), jnp.int32),
        ],
    )(ids_pad, counts)


def _gather_body(inv_hbm, h_hbm, m_hbm, hs_hbm, ms_hbm,
                 idxb, hbuf, mbuf, idxrem, hrem, mrem, isem, lsem, ssem):
    w = _wid()
    gbase = w * NGRP

    def valid(s):
        return (s >= 0) & (s < NGRP) & (gbase + s < G_FULL)

    def body(slot, _):
        s1 = slot - 1
        s3 = slot - 3

        @pl.when(valid(s3))
        def _():
            pltpu.make_async_copy(
                hbuf.at[s3 % 3], hs_hbm.at[idxb.at[s3 % 3]], ssem).wait()
            pltpu.make_async_copy(
                mbuf.at[s3 % 3], ms_hbm.at[idxb.at[s3 % 3]], ssem).wait()

        @pl.when(valid(slot))
        def _():
            off = (gbase + slot) * 128
            pltpu.async_copy(
                inv_hbm.at[pl.ds(off, 128)], idxb.at[slot % 3], isem)
            pltpu.async_copy(
                h_hbm.at[pl.ds(off, 128)], hbuf.at[slot % 3], lsem)
            pltpu.async_copy(
                m_hbm.at[pl.ds(off, 128)], mbuf.at[slot % 3], lsem)

        @pl.when(valid(s1))
        def _():
            off = (gbase + s1) * 128
            pltpu.make_async_copy(
                inv_hbm.at[pl.ds(off, 128)], idxb.at[s1 % 3], isem).wait()
            pltpu.make_async_copy(
                h_hbm.at[pl.ds(off, 128)], hbuf.at[s1 % 3], lsem).wait()
            pltpu.make_async_copy(
                m_hbm.at[pl.ds(off, 128)], mbuf.at[s1 % 3], lsem).wait()
            pltpu.async_copy(hbuf.at[s1 % 3], hs_hbm.at[idxb.at[s1 % 3]], ssem)
            pltpu.async_copy(mbuf.at[s1 % 3], ms_hbm.at[idxb.at[s1 % 3]], ssem)

        return 0

    lax.fori_loop(0, NGRP + 4, body, 0)

    @pl.when(w == 0)
    def _():
        pltpu.sync_copy(inv_hbm.at[pl.ds(REM_OFF, REM)], idxrem)
        pltpu.sync_copy(h_hbm.at[pl.ds(REM_OFF, REM)], hrem)
        pltpu.sync_copy(m_hbm.at[pl.ds(REM_OFF, REM)], mrem)
        pltpu.async_copy(hrem, hs_hbm.at[idxrem], ssem).wait()
        pltpu.async_copy(mrem, ms_hbm.at[idxrem], ssem).wait()


def _gather(inv, h, message):
    return pl.kernel(
        _gather_body,
        out_type=[
            jax.ShapeDtypeStruct((N, D), jnp.float32),
            jax.ShapeDtypeStruct((N, D), jnp.float32),
        ],
        mesh=plsc.VectorSubcoreMesh(**_MESH),
        name="sc_gather",
        scratch_types=[
            pltpu.VMEM((3, 128), jnp.int32),
            pltpu.VMEM((3, 128, D), jnp.float32),
            pltpu.VMEM((3, 128, D), jnp.float32),
            pltpu.VMEM((REM,), jnp.int32),
            pltpu.VMEM((REM, D), jnp.float32),
            pltpu.VMEM((REM, D), jnp.float32),
            pltpu.SemaphoreType.DMA,
            pltpu.SemaphoreType.DMA,
            pltpu.SemaphoreType.DMA,
        ],
    )(inv, h, message)


def _scatter_body(inv_hbm, outs_hbm, dh_hbm, idxb, rbuf, idxrem, rrem,
                  isem, gsem, wsem):
    w = _wid()
    gbase = w * NGRP

    def valid(s):
        return (s >= 0) & (s < NGRP) & (gbase + s < G_FULL)

    def body(slot, _):
        s1 = slot - 1
        s2 = slot - 2
        s4 = slot - 4

        @pl.when(valid(s4))
        def _():
            off = (gbase + s4) * 128
            pltpu.make_async_copy(
                rbuf.at[s4 % 3], dh_hbm.at[pl.ds(off, 128)], wsem).wait()

        @pl.when(valid(slot))
        def _():
            off = (gbase + slot) * 128
            pltpu.async_copy(
                inv_hbm.at[pl.ds(off, 128)], idxb.at[slot % 3], isem)

        @pl.when(valid(s1))
        def _():
            off = (gbase + s1) * 128
            pltpu.make_async_copy(
                inv_hbm.at[pl.ds(off, 128)], idxb.at[s1 % 3], isem).wait()
            pltpu.async_copy(
                outs_hbm.at[idxb.at[s1 % 3]], rbuf.at[s1 % 3], gsem)

        @pl.when(valid(s2))
        def _():
            off = (gbase + s2) * 128
            pltpu.make_async_copy(
                outs_hbm.at[idxb.at[s2 % 3]], rbuf.at[s2 % 3], gsem).wait()
            pltpu.async_copy(
                rbuf.at[s2 % 3], dh_hbm.at[pl.ds(off, 128)], wsem)

        return 0

    lax.fori_loop(0, NGRP + 5, body, 0)

    @pl.when(w == 0)
    def _():
        pltpu.sync_copy(inv_hbm.at[pl.ds(REM_OFF, REM)], idxrem)
        pltpu.async_copy(outs_hbm.at[idxrem], rrem, gsem).wait()
        pltpu.sync_copy(rrem, dh_hbm.at[pl.ds(REM_OFF, REM)])


def _scatter(inv, outs):
    return pl.kernel(
        _scatter_body,
        out_type=jax.ShapeDtypeStruct((N, D), jnp.float32),
        mesh=plsc.VectorSubcoreMesh(**_MESH),
        name="sc_scatter",
        scratch_types=[
            pltpu.VMEM((3, 128), jnp.int32),
            pltpu.VMEM((3, 128, D), jnp.float32),
            pltpu.VMEM((REM,), jnp.int32),
            pltpu.VMEM((REM, D), jnp.float32),
            pltpu.SemaphoreType.DMA,
            pltpu.SemaphoreType.DMA,
            pltpu.SemaphoreType.DMA,
        ],
    )(inv, outs)


def _sn(W):
    R = W.shape[0]
    u = jnp.full((R, 1), 1.0 / jnp.sqrt(jnp.float32(R)), dtype=jnp.float32)
    v = None
    for _ in range(7):
        v = jnp.sum(W * u, axis=0, keepdims=True)
        v = v / (jnp.sqrt(jnp.sum(v * v)) + 1e-12)
        u = jnp.sum(W * v, axis=1, keepdims=True)
        u = u / (jnp.sqrt(jnp.sum(u * u)) + 1e-12)
    Wv = jnp.sum(W * v, axis=1, keepdims=True)
    sigma = jnp.sum(u * Wv)
    return W / sigma


def _norm_body(w1_ref, w2_ref, w1o_ref, w2o_ref):
    for i in range(N_TYPES):
        w1o_ref[i] = _sn(w1_ref[i])
        w2o_ref[i] = _sn(w2_ref[i])


def _normalize(W1, W2, interpret=False):
    return pl.pallas_call(
        _norm_body,
        grid=(1,),
        in_specs=[
            pl.BlockSpec((N_TYPES, 2 * D, D), lambda i: (0, 0, 0)),
            pl.BlockSpec((N_TYPES, D, D), lambda i: (0, 0, 0)),
        ],
        out_specs=[
            pl.BlockSpec((N_TYPES, 2 * D, D), lambda i: (0, 0, 0)),
            pl.BlockSpec((N_TYPES, D, D), lambda i: (0, 0, 0)),
        ],
        out_shape=[
            jax.ShapeDtypeStruct((N_TYPES, 2 * D, D), jnp.float32),
            jax.ShapeDtypeStruct((N_TYPES, D, D), jnp.float32),
        ],
        interpret=interpret,
    )(W1, W2)


def _seg_body(bases_ref, hs_ref, ms_ref, w1_ref, b1_ref, w2_ref, b2_ref,
              out_ref):
    b = pl.program_id(0)
    start = b * BLK
    hx = hs_ref[...]
    mx = ms_ref[...]
    ridx = lax.broadcasted_iota(jnp.int32, (BLK, 1), 0) + start
    out_ref[...] = jnp.zeros((BLK, D), jnp.float32)
    for t in range(N_TYPES):
        lo = bases_ref[t]
        hi = bases_ref[t + 1]

        @pl.when((lo < start + BLK) & (hi > start))
        def _():
            y = (
                jnp.dot(hx, w1_ref[t, :D, :],
                        preferred_element_type=jnp.float32)
                + jnp.dot(mx, w1_ref[t, D:, :],
                          preferred_element_type=jnp.float32)
                + b1_ref[t]
            )
            y = y * jax.nn.sigmoid(y)
            z = jnp.dot(y, w2_ref[t], preferred_element_type=jnp.float32)
            z = z + b2_ref[t]
            mask = ((ridx >= lo) & (ridx < hi)).astype(jnp.float32)
            out_ref[...] += z * mask


def _seg_mlp(bases, hs, ms, W1n, b1, W2n, b2):
    grid_spec = pltpu.PrefetchScalarGridSpec(
        num_scalar_prefetch=1,
        grid=(N_BLOCKS,),
        in_specs=[
            pl.BlockSpec((BLK, D), lambda i, s: (i, 0)),
            pl.BlockSpec((BLK, D), lambda i, s: (i, 0)),
            pl.BlockSpec((N_TYPES, 2 * D, D), lambda i, s: (0, 0, 0)),
            pl.BlockSpec((N_TYPES, D), lambda i, s: (0, 0)),
            pl.BlockSpec((N_TYPES, D, D), lambda i, s: (0, 0, 0)),
            pl.BlockSpec((N_TYPES, D), lambda i, s: (0, 0)),
        ],
        out_specs=pl.BlockSpec((BLK, D), lambda i, s: (i, 0)),
    )
    return pl.pallas_call(
        _seg_body,
        grid_spec=grid_spec,
        out_shape=jax.ShapeDtypeStruct((N, D), jnp.float32),
        compiler_params=pltpu.CompilerParams(
            dimension_semantics=("arbitrary",),
        ),
    )(bases, hs, ms, W1n, b1, W2n, b2)


def kernel(h, message, node_type_ids, W1, b1, W2, b2):
    ids = node_type_ids.astype(jnp.int32)
    ids_pad = jnp.concatenate(
        [ids, jnp.full((NP - N,), SENTINEL, jnp.int32)])
    counts = _hist(ids_pad)
    inv, bases = _dispatch(ids_pad, counts)
    W1n, W2n = _normalize(W1, W2)
    hs, ms = _gather(inv, h, message)
    outs = _seg_mlp(bases, hs, ms, W1n, b1, W2n, b2)
    dh = _scatter(inv, outs)
    return dh


# speedup vs baseline: 1.0989x; 1.0581x over previous
import functools

import jax
import jax.numpy as jnp
from jax import lax
from jax.experimental import pallas as pl
from jax.experimental.pallas import tpu as pltpu
from jax.experimental.pallas import tpu_sc as plsc

N_TYPES = 8
D = 128
N = 100000
BLK = 4000
N_BLOCKS = N // BLK

NW = 32
NP = 102400
CH = NP // NW
NVREG = CH // 16
NGRP = CH // 128
G_FULL = N // 128
REM = N - G_FULL * 128
REM_OFF = G_FULL * 128
SENTINEL = 127

_MESH = dict(core_axis_name="c", subcore_axis_name="s")


def _wid():
    return lax.axis_index("s") * 2 + lax.axis_index("c")


def _hist_body(ids_hbm, counts_hbm, idsv, cntv):
    w = _wid()
    pltpu.sync_copy(ids_hbm.at[pl.ds(w * CH, CH)], idsv)
    lane = lax.iota(jnp.int32, 16)
    zeros = jnp.zeros((16,), jnp.int32)

    one = jnp.ones((16,), jnp.int32)

    def body(i, cs):
        vec = idsv[pl.ds(i * 16, 16)]
        return tuple(
            cs[t] + jnp.where(vec == t, one, zeros) for t in range(N_TYPES)
        )

    cs = lax.fori_loop(0, NVREG, body, (zeros,) * N_TYPES)
    cv = zeros
    for t in range(N_TYPES):
        tot = jnp.int32(0)
        for l in range(16):
            tot = tot + cs[t][l]
        cv = jnp.where(lane == t, tot, cv)
    cntv[...] = cv
    pltpu.sync_copy(cntv, counts_hbm.at[w])


def _hist(ids_pad):
    return pl.kernel(
        _hist_body,
        out_type=jax.ShapeDtypeStruct((NW, 16), jnp.int32),
        mesh=plsc.VectorSubcoreMesh(**_MESH),
        name="sc_hist",
        scratch_types=[
            pltpu.VMEM((CH,), jnp.int32),
            pltpu.VMEM((16,), jnp.int32),
        ],
    )(ids_pad)


def _dispatch_body(ids_hbm, counts_hbm, inv_hbm, bases_hbm,
                   idsv, cntall, destv, basev):
    w = _wid()
    pltpu.sync_copy(ids_hbm.at[pl.ds(w * CH, CH)], idsv)
    pltpu.sync_copy(counts_hbm, cntall)
    lane = lax.iota(jnp.int32, 16)
    zeros = jnp.zeros((16,), jnp.int32)

    rows = [cntall[wp] for wp in range(NW)]
    base = jnp.int32(0)
    bv = zeros
    offs0 = []
    for t in range(N_TYPES):
        myprefix = jnp.int32(0)
        total = jnp.int32(0)
        for wp in range(NW):
            c = rows[wp][t]
            total = total + c
            myprefix = myprefix + jnp.where(jnp.int32(wp) < w, c, 0)
        bv = jnp.where(lane == t, base, bv)
        offs0.append(base + myprefix)
        base = base + total
    offs0.append(jnp.int32(N))
    bv = jnp.where(lane >= N_TYPES, jnp.int32(N), bv)
    basev[...] = bv

    @pl.when(w == 0)
    def _():
        pltpu.sync_copy(basev, bases_hbm)

    one = jnp.ones((16,), jnp.int32)
    lane_gt = [lane > k for k in range(16)]

    def body(i, offs):
        vec = idsv[pl.ds(i * 16, 16)]
        tidv = jnp.minimum(vec, N_TYPES)
        tids = [tidv[k] for k in range(16)]
        r = zeros
        for k in range(16):
            r = r + jnp.where(lane_gt[k] & (tidv == tids[k]), one, zeros)
        offv = zeros
        for t in range(N_TYPES + 1):
            offv = offv + jnp.where(tidv == t, offs[t], 0)
        destv[pl.ds(i * 16, 16)] = offv + r
        new_offs = []
        for t in range(N_TYPES + 1):
            c = jnp.int32(0)
            for k in range(16):
                c = c + jnp.where(tids[k] == t, 1, 0)
            new_offs.append(offs[t] + c)
        return tuple(new_offs)

    lax.fori_loop(0, NVREG, body, tuple(offs0))
    pltpu.sync_copy(destv, inv_hbm.at[pl.ds(w * CH, CH)])


def _dispatch(ids_pad, counts):
    return pl.kernel(
        _dispatch_body,
        out_type=[
            jax.ShapeDtypeStruct((NP,), jnp.int32),
            jax.ShapeDtypeStruct((16,), jnp.int32),
        ],
        mesh=plsc.VectorSubcoreMesh(**_MESH),
        name="sc_dispatch",
        scratch_types=[
            pltpu.VMEM((CH,), jnp.int32),
            pltpu.VMEM((NW, 16), jnp.int32),
            pltpu.VMEM((CH,), jnp.int32),
            pltpu.VMEM((16,), jnp.int32),
        ],
    )(ids_pad, counts)


def _gather_body(inv_hbm, xq_hbm, xqs_hbm,
                 idxb, xbuf, idxrem, xrem, isem, lsem, ssem):
    w = _wid()
    gbase = w * NGRP

    def valid(s):
        return (s >= 0) & (s < NGRP) & (gbase + s < G_FULL)

    def body(slot, _):
        s1 = slot - 1
        s3 = slot - 3

        @pl.when(valid(s3))
        def _():
            pltpu.make_async_copy(
                xbuf.at[s3 % 3], xqs_hbm.at[idxb.at[s3 % 3]], ssem).wait()

        @pl.when(valid(slot))
        def _():
            off = (gbase + slot) * 128
            pltpu.async_copy(
                inv_hbm.at[pl.ds(off, 128)], idxb.at[slot % 3], isem)
            pltpu.async_copy(
                xq_hbm.at[pl.ds(off, 128)], xbuf.at[slot % 3], lsem)

        @pl.when(valid(s1))
        def _():
            off = (gbase + s1) * 128
            pltpu.make_async_copy(
                inv_hbm.at[pl.ds(off, 128)], idxb.at[s1 % 3], isem).wait()
            pltpu.make_async_copy(
                xq_hbm.at[pl.ds(off, 128)], xbuf.at[s1 % 3], lsem).wait()
            pltpu.async_copy(xbuf.at[s1 % 3], xqs_hbm.at[idxb.at[s1 % 3]], ssem)

        return 0

    lax.fori_loop(0, NGRP + 4, body, 0)

    @pl.when(w == 0)
    def _():
        pltpu.sync_copy(inv_hbm.at[pl.ds(REM_OFF, REM)], idxrem)
        pltpu.sync_copy(xq_hbm.at[pl.ds(REM_OFF, REM)], xrem)
        pltpu.async_copy(xrem, xqs_hbm.at[idxrem], ssem).wait()


def _gather(inv, xq):
    return pl.kernel(
        _gather_body,
        out_type=jax.ShapeDtypeStruct((N, D), jnp.int32),
        mesh=plsc.VectorSubcoreMesh(**_MESH),
        name="sc_gather",
        scratch_types=[
            pltpu.VMEM((3, 128), jnp.int32),
            pltpu.VMEM((3, 128, D), jnp.int32),
            pltpu.VMEM((REM,), jnp.int32),
            pltpu.VMEM((REM, D), jnp.int32),
            pltpu.SemaphoreType.DMA,
            pltpu.SemaphoreType.DMA,
            pltpu.SemaphoreType.DMA,
        ],
    )(inv, xq)


def _scatter_body(inv_hbm, outs_hbm, dh_hbm, idxb, rbuf, idxrem, rrem,
                  isem, gsem, wsem):
    w = _wid()
    gbase = w * NGRP

    def valid(s):
        return (s >= 0) & (s < NGRP) & (gbase + s < G_FULL)

    def body(slot, _):
        s1 = slot - 1
        s2 = slot - 2
        s4 = slot - 4

        @pl.when(valid(s4))
        def _():
            off = (gbase + s4) * 128
            pltpu.make_async_copy(
                rbuf.at[s4 % 3], dh_hbm.at[pl.ds(off, 128)], wsem).wait()

        @pl.when(valid(slot))
        def _():
            off = (gbase + slot) * 128
            pltpu.async_copy(
                inv_hbm.at[pl.ds(off, 128)], idxb.at[slot % 3], isem)

        @pl.when(valid(s1))
        def _():
            off = (gbase + s1) * 128
            pltpu.make_async_copy(
                inv_hbm.at[pl.ds(off, 128)], idxb.at[s1 % 3], isem).wait()
            pltpu.async_copy(
                outs_hbm.at[idxb.at[s1 % 3]], rbuf.at[s1 % 3], gsem)

        @pl.when(valid(s2))
        def _():
            off = (gbase + s2) * 128
            pltpu.make_async_copy(
                outs_hbm.at[idxb.at[s2 % 3]], rbuf.at[s2 % 3], gsem).wait()
            pltpu.async_copy(
                rbuf.at[s2 % 3], dh_hbm.at[pl.ds(off, 128)], wsem)

        return 0

    lax.fori_loop(0, NGRP + 5, body, 0)

    @pl.when(w == 0)
    def _():
        pltpu.sync_copy(inv_hbm.at[pl.ds(REM_OFF, REM)], idxrem)
        pltpu.async_copy(outs_hbm.at[idxrem], rrem, gsem).wait()
        pltpu.sync_copy(rrem, dh_hbm.at[pl.ds(REM_OFF, REM)])


def _scatter(inv, outs):
    return pl.kernel(
        _scatter_body,
        out_type=jax.ShapeDtypeStruct((N, D), jnp.float32),
        mesh=plsc.VectorSubcoreMesh(**_MESH),
        name="sc_scatter",
        scratch_types=[
            pltpu.VMEM((3, 128), jnp.int32),
            pltpu.VMEM((3, 128, D), jnp.float32),
            pltpu.VMEM((REM,), jnp.int32),
            pltpu.VMEM((REM, D), jnp.float32),
            pltpu.SemaphoreType.DMA,
            pltpu.SemaphoreType.DMA,
            pltpu.SemaphoreType.DMA,
        ],
    )(inv, outs)


def _pack_body(h_ref, m_ref, xq_ref):
    h16 = lax.bitcast_convert_type(
        h_ref[...].astype(jnp.bfloat16), jnp.uint16).astype(jnp.int32)
    m16 = lax.bitcast_convert_type(
        m_ref[...].astype(jnp.bfloat16), jnp.uint16).astype(jnp.int32)
    xq_ref[...] = h16 | (m16 << 16)


def _pack(h, message):
    return pl.pallas_call(
        _pack_body,
        grid=(N_BLOCKS,),
        in_specs=[
            pl.BlockSpec((BLK, D), lambda i: (i, 0)),
            pl.BlockSpec((BLK, D), lambda i: (i, 0)),
        ],
        out_specs=pl.BlockSpec((BLK, D), lambda i: (i, 0)),
        out_shape=jax.ShapeDtypeStruct((N, D), jnp.int32),
    )(h, message)


def _sn(W):
    R = W.shape[0]
    u = jnp.full((R, 1), 1.0 / jnp.sqrt(jnp.float32(R)), dtype=jnp.float32)
    v = None
    for _ in range(7):
        v = jnp.sum(W * u, axis=0, keepdims=True)
        v = v / (jnp.sqrt(jnp.sum(v * v)) + 1e-12)
        u = jnp.sum(W * v, axis=1, keepdims=True)
        u = u / (jnp.sqrt(jnp.sum(u * u)) + 1e-12)
    Wv = jnp.sum(W * v, axis=1, keepdims=True)
    sigma = jnp.sum(u * Wv)
    return W / sigma


def _norm_body(w1_ref, w2_ref, w1o_ref, w2o_ref):
    for i in range(N_TYPES):
        w1o_ref[i] = _sn(w1_ref[i])
        w2o_ref[i] = _sn(w2_ref[i])


def _normalize(W1, W2, interpret=False):
    return pl.pallas_call(
        _norm_body,
        grid=(1,),
        in_specs=[
            pl.BlockSpec((N_TYPES, 2 * D, D), lambda i: (0, 0, 0)),
            pl.BlockSpec((N_TYPES, D, D), lambda i: (0, 0, 0)),
        ],
        out_specs=[
            pl.BlockSpec((N_TYPES, 2 * D, D), lambda i: (0, 0, 0)),
            pl.BlockSpec((N_TYPES, D, D), lambda i: (0, 0, 0)),
        ],
        out_shape=[
            jax.ShapeDtypeStruct((N_TYPES, 2 * D, D), jnp.float32),
            jax.ShapeDtypeStruct((N_TYPES, D, D), jnp.float32),
        ],
        interpret=interpret,
    )(W1, W2)


def _seg_body(bases_ref, xqs_ref, w1_ref, b1_ref, w2_ref, b2_ref,
              out_ref):
    b = pl.program_id(0)
    start = b * BLK
    x32 = xqs_ref[...]
    hx = lax.bitcast_convert_type(
        (x32 & 0xFFFF).astype(jnp.uint16), jnp.bfloat16)
    mx = lax.bitcast_convert_type(
        lax.shift_right_logical(x32, 16).astype(jnp.uint16), jnp.bfloat16)
    ridx = lax.broadcasted_iota(jnp.int32, (BLK, 1), 0) + start
    out_ref[...] = jnp.zeros((BLK, D), jnp.float32)
    for t in range(N_TYPES):
        lo = bases_ref[t]
        hi = bases_ref[t + 1]

        @pl.when((lo < start + BLK) & (hi > start))
        def _():
            y = (
                jnp.dot(hx, w1_ref[t, :D, :].astype(jnp.bfloat16),
                        preferred_element_type=jnp.float32)
                + jnp.dot(mx, w1_ref[t, D:, :].astype(jnp.bfloat16),
                          preferred_element_type=jnp.float32)
                + b1_ref[t]
            )
            y = y * jax.nn.sigmoid(y)
            z = jnp.dot(y.astype(jnp.bfloat16),
                        w2_ref[t].astype(jnp.bfloat16),
                        preferred_element_type=jnp.float32)
            z = z + b2_ref[t]
            mask = ((ridx >= lo) & (ridx < hi)).astype(jnp.float32)
            out_ref[...] += z * mask


def _seg_mlp(bases, xqs, W1n, b1, W2n, b2):
    grid_spec = pltpu.PrefetchScalarGridSpec(
        num_scalar_prefetch=1,
        grid=(N_BLOCKS,),
        in_specs=[
            pl.BlockSpec((BLK, D), lambda i, s: (i, 0)),
            pl.BlockSpec((N_TYPES, 2 * D, D), lambda i, s: (0, 0, 0)),
            pl.BlockSpec((N_TYPES, D), lambda i, s: (0, 0)),
            pl.BlockSpec((N_TYPES, D, D), lambda i, s: (0, 0, 0)),
            pl.BlockSpec((N_TYPES, D), lambda i, s: (0, 0)),
        ],
        out_specs=pl.BlockSpec((BLK, D), lambda i, s: (i, 0)),
    )
    return pl.pallas_call(
        _seg_body,
        grid_spec=grid_spec,
        out_shape=jax.ShapeDtypeStruct((N, D), jnp.float32),
        compiler_params=pltpu.CompilerParams(
            dimension_semantics=("arbitrary",),
        ),
    )(bases, xqs, W1n, b1, W2n, b2)


def kernel(h, message, node_type_ids, W1, b1, W2, b2):
    ids = node_type_ids.astype(jnp.int32)
    ids_pad = jnp.concatenate(
        [ids, jnp.full((NP - N,), SENTINEL, jnp.int32)])
    xq = _pack(h, message)
    counts = _hist(ids_pad)
    inv, bases = _dispatch(ids_pad, counts)
    W1n, W2n = _normalize(W1, W2)
    xqs = _gather(inv, xq)
    outs = _seg_mlp(bases, xqs, W1n, b1, W2n, b2)
    dh = _scatter(inv, outs)
    return dh
